# Initial kernel scaffold; baseline (speedup 1.0000x reference)
import jax
import jax.numpy as jnp
from jax.experimental import pallas as pl


def kernel(h, edge_index, edge_weight, W_self, W_msg, bias, gamma, beta):
    raise NotImplementedError("write your pallas kernel here")


# speedup vs baseline: 2.6426x; 2.6426x over previous
import functools

import jax
import jax.numpy as jnp
from jax import lax
from jax.experimental import pallas as pl
from jax.experimental.pallas import tpu as pltpu
from jax.experimental.pallas import tpu_sc as plsc

_N = 10000
_D = 256
_E = 160000
_HALF = _D // 2
_NS = 16
_EP = _E // _NS
_CH = 80
_NCH = _EP // _CH
_GP = _CH // 16
_RBASE = 624
_ZROWS = 48


def _sc_segment_sum(h_lo, h_hi, src, dst, ew):
    mesh = plsc.VectorSubcoreMesh(core_axis_name="c", subcore_axis_name="s")

    @functools.partial(
        pl.kernel,
        mesh=mesh,
        out_type=(
            jax.ShapeDtypeStruct((2, _N, _HALF), jnp.float32),
            jax.ShapeDtypeStruct((_N,), jnp.float32),
        ),
        scratch_types=[
            pltpu.VMEM((_CH + 16,), jnp.float32),
            pltpu.VMEM((_CH,), jnp.int32),
            pltpu.VMEM((_CH,), jnp.int32),
            pltpu.VMEM((_CH, _HALF), jnp.float32),
            pltpu.VMEM((_N + 16,), jnp.float32),
            pltpu.VMEM((_RBASE + 16,), jnp.float32),
            pltpu.VMEM((_ZROWS, _HALF), jnp.float32),
            pltpu.VMEM_SHARED((_N, _HALF), jnp.float32),
            pltpu.VMEM_SHARED((_NS * _N,), jnp.float32),
        ],
    )
    def sc_kernel(hlo, hhi, srcx, dstx, eww, acc_out, ws_out,
                  w_b, src_b, dst_b, rows_v, ws_loc, ws_red, zbuf,
                  acc_sh, ws_all):
        c = lax.axis_index("c")
        s = lax.axis_index("s")
        base_e = s * _EP
        row0 = pl.multiple_of(_RBASE * s + 16 * jnp.minimum(s, 1), 16)

        def _zw(i, carry):
            ws_loc[pl.ds(i * 16, 16)] = jnp.zeros((16,), jnp.float32)
            return carry
        lax.fori_loop(0, (_N + 16) // 16, _zw, 0)

        def _zb(i, carry):
            r = i // (_HALF // 16)
            j = i % (_HALF // 16)
            zbuf[r, pl.ds(j * 16, 16)] = jnp.zeros((16,), jnp.float32)
            return carry
        lax.fori_loop(0, _ZROWS * (_HALF // 16), _zb, 0)
        for t in range(_RBASE // _ZROWS):
            pltpu.sync_copy(
                zbuf,
                acc_sh.at[pl.ds(pl.multiple_of(row0 + t * _ZROWS, 8),
                                _ZROWS)])

        @pl.when(s == 0)
        def _():
            pltpu.sync_copy(zbuf.at[pl.ds(0, 16)],
                            acc_sh.at[pl.ds(_RBASE, 16)])

        plsc.subcore_barrier()

        lanes = lax.iota(jnp.int32, 16)

        def _chunk(k, carry):
            off = base_e + k * _CH
            pltpu.sync_copy(srcx.at[pl.ds(off, _CH)], src_b)
            pltpu.sync_copy(dstx.at[pl.ds(off, _CH)], dst_b)
            pltpu.sync_copy(eww.at[pl.ds(off, _CH)], w_b.at[pl.ds(0, _CH)])

            def _sig(i, c2):
                v = w_b[pl.ds(i * 16, 16)]
                w_b[pl.ds(i * 16, 16)] = 1.0 / (1.0 + jnp.exp(-v))
                return c2
            lax.fori_loop(0, _CH // 16, _sig, 0)

            @pl.when(c == 0)
            def _():
                pltpu.sync_copy(hlo.at[src_b], rows_v)

            @pl.when(c == 1)
            def _():
                pltpu.sync_copy(hhi.at[src_b], rows_v)

            def _group(g, carry2):
                w16 = w_b[pl.ds(g * 16, 16)]
                d16 = dst_b[pl.ds(g * 16, 16)]
                for lane in range(16):
                    wl = w16[lane]
                    e = g * 16 + lane
                    for j in range(_HALF // 16):
                        rows_v[e, pl.ds(j * 16, 16)] = (
                            rows_v[e, pl.ds(j * 16, 16)] * wl)
                    d = d16[lane]
                    woff = (d // 16) * 16
                    lv = d - woff
                    win = ws_loc[pl.ds(woff, 16)]
                    ws_loc[pl.ds(woff, 16)] = win + jnp.where(
                        lanes == lv, wl, jnp.float32(0.0))
                return carry2
            lax.fori_loop(0, _GP, _group, 0)

            pltpu.sync_copy(rows_v, acc_sh.at[dst_b], add=True)
            return carry
        lax.fori_loop(0, _NCH, _chunk, 0)

        pltpu.sync_copy(ws_loc.at[pl.ds(0, _N)],
                        ws_all.at[pl.ds(s * _N, _N)])
        plsc.subcore_barrier()

        nred = _RBASE // 16

        def _zr(i, carry):
            ws_red[pl.ds(i * 16, 16)] = jnp.zeros((16,), jnp.float32)
            return carry
        lax.fori_loop(0, (_RBASE + 16) // 16, _zr, 0)

        for t in range(_NS):
            pltpu.sync_copy(
                ws_all.at[pl.ds(t * _N + row0, _RBASE)],
                ws_loc.at[pl.ds(0, _RBASE)])

            @pl.when(s == 0)
            def _():
                pltpu.sync_copy(ws_all.at[pl.ds(t * _N + _RBASE, 16)],
                                ws_loc.at[pl.ds(_RBASE, 16)])

            def _acc(i, carry):
                ws_red[pl.ds(i * 16, 16)] = (
                    ws_red[pl.ds(i * 16, 16)] + ws_loc[pl.ds(i * 16, 16)])
                return carry
            lax.fori_loop(0, nred, _acc, 0)

            @pl.when(s == 0)
            def _():
                ws_red[pl.ds(_RBASE, 16)] = (
                    ws_red[pl.ds(_RBASE, 16)] + ws_loc[pl.ds(_RBASE, 16)])

        @pl.when(c == 0)
        def _():
            pltpu.sync_copy(ws_red.at[pl.ds(0, _RBASE)],
                            ws_out.at[pl.ds(row0, _RBASE)])

            @pl.when(s == 0)
            def _():
                pltpu.sync_copy(ws_red.at[pl.ds(_RBASE, 16)],
                                ws_out.at[pl.ds(_RBASE, 16)])

        pltpu.sync_copy(acc_sh.at[pl.ds(row0, _RBASE)],
                        acc_out.at[c].at[pl.ds(row0, _RBASE)])

        @pl.when(s == 0)
        def _():
            pltpu.sync_copy(acc_sh.at[pl.ds(_RBASE, 16)],
                            acc_out.at[c].at[pl.ds(_RBASE, 16)])

    return sc_kernel(h_lo, h_hi, src, dst, ew)


def _tc_body(x_ref, acc_ref, ws_ref, wst_ref, wmt_ref, b_ref, g_ref, be_ref,
             o_ref):
    x = x_ref[...]
    hs = jnp.dot(x, wst_ref[...], preferred_element_type=jnp.float32)
    m = jnp.dot(acc_ref[...], wmt_ref[...], preferred_element_type=jnp.float32)
    ws = jnp.maximum(ws_ref[...], 1e-8)
    pre = jnp.maximum(hs + m / ws + b_ref[...], 0.0)
    y = x + pre
    mu = jnp.mean(y, axis=1, keepdims=True)
    var = jnp.mean((y - mu) ** 2, axis=1, keepdims=True)
    o_ref[...] = ((y - mu) * lax.rsqrt(var + 1e-5) * g_ref[...] + be_ref[...])


def _tc_dense(x2, acc, wsum, wst, wmt, bias, gamma, beta):
    R = 1000
    grid = (_N // R,)
    return pl.pallas_call(
        _tc_body,
        grid=grid,
        in_specs=[
            pl.BlockSpec((R, _D), lambda i: (i, 0)),
            pl.BlockSpec((R, _D), lambda i: (i, 0)),
            pl.BlockSpec((R, 1), lambda i: (i, 0)),
            pl.BlockSpec((_D, _D), lambda i: (0, 0)),
            pl.BlockSpec((_D, _D), lambda i: (0, 0)),
            pl.BlockSpec((1, _D), lambda i: (0, 0)),
            pl.BlockSpec((1, _D), lambda i: (0, 0)),
            pl.BlockSpec((1, _D), lambda i: (0, 0)),
        ],
        out_specs=pl.BlockSpec((R, _D), lambda i: (i, 0)),
        out_shape=jax.ShapeDtypeStruct((_N, _D), jnp.float32),
    )(x2, acc, wsum, wst, wmt, bias, gamma, beta)


def kernel(h, edge_index, edge_weight, W_self, W_msg, bias, gamma, beta):
    B, N, D = h.shape
    x2 = h.reshape(N, D)
    h_lo = x2[:, :_HALF]
    h_hi = x2[:, _HALF:]
    src = edge_index[0]
    dst = edge_index[1]

    accw, wsum = _sc_segment_sum(h_lo, h_hi, src, dst, edge_weight)
    acc = jnp.concatenate([accw[0], accw[1]], axis=1)

    out = _tc_dense(x2, acc, wsum.reshape(N, 1), W_self.T, W_msg.T,
                    bias.reshape(1, D), gamma.reshape(1, D),
                    beta.reshape(1, D))
    return out.reshape(B, N, D)


# speedup vs baseline: 5.1358x; 1.9434x over previous
import functools

import jax
import jax.numpy as jnp
from jax import lax
from jax.experimental import pallas as pl
from jax.experimental.pallas import tpu as pltpu
from jax.experimental.pallas import tpu_sc as plsc

_N = 10000
_D = 256
_E = 160000
_HALF = _D // 2
_NS = 16
_EP = _E // _NS
_CH = 80
_NCH = _EP // _CH
_GP = _CH // 16
_RBASE = 624
_ZROWS = 48
_WROWS = 80


def _sc_segment_sum(h_lo, h_hi, src, dst, ew):
    mesh = plsc.VectorSubcoreMesh(core_axis_name="c", subcore_axis_name="s")

    @functools.partial(
        pl.kernel,
        mesh=mesh,
        out_type=(
            jax.ShapeDtypeStruct((2, _N, _HALF), jnp.float32),
            jax.ShapeDtypeStruct((_WROWS, _HALF), jnp.float32),
        ),
        scratch_types=[
            pltpu.VMEM((_EP + 16,), jnp.float32),
            pltpu.VMEM((_CH, _HALF), jnp.float32),
            pltpu.VMEM((_CH, _HALF), jnp.float32),
            pltpu.VMEM((_WROWS, _HALF), jnp.float32),
            pltpu.VMEM((_ZROWS, _HALF), jnp.float32),
            pltpu.VMEM((_CH,), jnp.int32),
            pltpu.VMEM((_CH,), jnp.int32),
            pltpu.VMEM((_CH,), jnp.int32),
            pltpu.VMEM((_CH,), jnp.int32),
            pltpu.VMEM((_CH,), jnp.int32),
            pltpu.VMEM((_CH,), jnp.int32),
            pltpu.VMEM((_WROWS,), jnp.int32),
            pltpu.SemaphoreType.DMA,
            pltpu.SemaphoreType.DMA,
            pltpu.SemaphoreType.DMA,
            pltpu.SemaphoreType.DMA,
            pltpu.SemaphoreType.DMA,
            pltpu.SemaphoreType.DMA,
            pltpu.VMEM_SHARED((_N, _HALF), jnp.float32),
            pltpu.VMEM_SHARED((_WROWS, _HALF), jnp.float32),
        ],
    )
    def sc_kernel(hlo, hhi, srcx, dstx, eww, acc_out, ws_out,
                  w_v, rows0, rows1, wsl, zbuf,
                  srcb0, srcb1, dstb0, dstb1, dstb2, dstb3, iota_b,
                  sidx0, sidx1, sg0, sg1, ss0, ss1,
                  acc_sh, ws_sh):
        rows = (rows0, rows1)
        srcb = (srcb0, srcb1)
        dstb = (dstb0, dstb1, dstb2, dstb3)
        sidx = (sidx0, sidx1)
        sg = (sg0, sg1)
        ss = (ss0, ss1)

        c = lax.axis_index("c")
        s = lax.axis_index("s")
        base_e = s * _EP
        row0 = pl.multiple_of(_RBASE * s + 16 * jnp.minimum(s, 1), 16)
        lanes = lax.iota(jnp.int32, 16)

        pltpu.sync_copy(eww.at[pl.ds(base_e, _EP)], w_v.at[pl.ds(0, _EP)])

        def _sig(i, carry):
            v = w_v[pl.ds(i * 16, 16)]
            w_v[pl.ds(i * 16, 16)] = 1.0 / (1.0 + jnp.exp(-v))
            return carry
        lax.fori_loop(0, _EP // 16, _sig, 0)

        def _zw(i, carry):
            r = i // (_HALF // 16)
            j = i % (_HALF // 16)
            wsl[r, pl.ds(j * 16, 16)] = jnp.zeros((16,), jnp.float32)
            return carry
        lax.fori_loop(0, _WROWS * (_HALF // 16), _zw, 0)

        def _zb(i, carry):
            r = i // (_HALF // 16)
            j = i % (_HALF // 16)
            zbuf[r, pl.ds(j * 16, 16)] = jnp.zeros((16,), jnp.float32)
            return carry
        lax.fori_loop(0, _ZROWS * (_HALF // 16), _zb, 0)

        def _io(i, carry):
            iota_b[pl.ds(i * 16, 16)] = lanes + i * 16
            return carry
        lax.fori_loop(0, _WROWS // 16, _io, 0)

        for t in range(_RBASE // _ZROWS):
            pltpu.sync_copy(
                zbuf,
                acc_sh.at[pl.ds(pl.multiple_of(row0 + t * _ZROWS, 8),
                                _ZROWS)])

        @pl.when(s == 0)
        def _():
            pltpu.sync_copy(zbuf.at[pl.ds(0, 16)],
                            acc_sh.at[pl.ds(_RBASE, 16)])
            pltpu.sync_copy(zbuf, ws_sh.at[pl.ds(0, _ZROWS)])
            pltpu.sync_copy(zbuf.at[pl.ds(0, _WROWS - _ZROWS)],
                            ws_sh.at[pl.ds(_ZROWS, _WROWS - _ZROWS)])

        plsc.subcore_barrier()

        def issue_idx(k, sslot, dslot):
            off = base_e + k * _CH
            pltpu.async_copy(srcx.at[pl.ds(off, _CH)], srcb[sslot],
                             sidx[sslot])
            pltpu.async_copy(dstx.at[pl.ds(off, _CH)], dstb[dslot],
                             sidx[sslot])

        def wait_idx(sslot, dslot):
            pltpu.make_async_copy(srcx.at[pl.ds(0, _CH)], srcb[sslot],
                                  sidx[sslot]).wait()
            pltpu.make_async_copy(dstx.at[pl.ds(0, _CH)], dstb[dslot],
                                  sidx[sslot]).wait()

        def issue_gather(b):
            @pl.when(c == 0)
            def _():
                pltpu.async_copy(hlo.at[srcb[b]], rows[b], sg[b])

            @pl.when(c == 1)
            def _():
                pltpu.async_copy(hhi.at[srcb[b]], rows[b], sg[b])

        def wait_gather(b):
            pltpu.make_async_copy(hlo.at[srcb[b]], rows[b], sg[b]).wait()

        def issue_scatter(b, dslot):
            pltpu.async_copy(rows[b], acc_sh.at[dstb[dslot]], ss[b],
                             add=True)

        def wait_scatter(b, dslot):
            pltpu.make_async_copy(rows[b], acc_sh.at[dstb[dslot]],
                                  ss[b]).wait()

        def compute(k, b, dslot):
            def _group(g, carry2):
                w16 = w_v[pl.ds(k * _CH + g * 16, 16)]
                d16 = dstb[dslot][pl.ds(g * 16, 16)]
                for lane in range(16):
                    wl = w16[lane]
                    e = g * 16 + lane
                    for j in range(_HALF // 16):
                        rows[b][e, pl.ds(j * 16, 16)] = (
                            rows[b][e, pl.ds(j * 16, 16)] * wl)
                    d = d16[lane]
                    woff = (d // 16) * 16
                    r = woff // _HALF
                    co = woff - r * _HALF
                    lv = d - woff
                    win = wsl[r, pl.ds(co, 16)]
                    wsl[r, pl.ds(co, 16)] = win + jnp.where(
                        lanes == lv, wl, jnp.float32(0.0))
                return carry2
            lax.fori_loop(0, _GP, _group, 0)

        def chunk_body(k, j, in_loop):
            b = j % 2
            b2 = (j + 1) % 2
            wait_gather(b)
            if in_loop:
                wait_idx(b2, (j + 1) % 4)

                @pl.when(k >= 1)
                def _():
                    wait_scatter(b2, (j + 3) % 4)
                issue_gather(b2)

                @pl.when(k + 2 < _NCH)
                def _():
                    issue_idx(k + 2, b, (j + 2) % 4)
            else:
                wait_scatter(b2, (j + 3) % 4)
            compute(k, b, j % 4)
            issue_scatter(b, j % 4)

        issue_idx(0, 0, 0)
        wait_idx(0, 0)
        issue_gather(0)
        issue_idx(1, 1, 1)

        def _quad(i, carry):
            for j in range(4):
                chunk_body(4 * i + j, j, True)
            return carry
        lax.fori_loop(0, (_NCH - 1) // 4, _quad, 0)

        chunk_body(_NCH - 1, 0, False)
        wait_scatter(0, 0)

        pltpu.sync_copy(wsl, ws_sh.at[iota_b], add=True)
        plsc.subcore_barrier()

        @pl.when(jnp.logical_and(c == 0, s == 0))
        def _():
            pltpu.sync_copy(ws_sh, ws_out)

        pltpu.sync_copy(acc_sh.at[pl.ds(row0, _RBASE)],
                        acc_out.at[c].at[pl.ds(row0, _RBASE)])

        @pl.when(s == 0)
        def _():
            pltpu.sync_copy(acc_sh.at[pl.ds(_RBASE, 16)],
                            acc_out.at[c].at[pl.ds(_RBASE, 16)])

    return sc_kernel(h_lo, h_hi, src, dst, ew)


def _tc_body(x_ref, acc_ref, ws_ref, wst_ref, wmt_ref, b_ref, g_ref, be_ref,
             o_ref):
    x = x_ref[...]
    hs = jnp.dot(x, wst_ref[...], preferred_element_type=jnp.float32)
    m = jnp.dot(acc_ref[...], wmt_ref[...], preferred_element_type=jnp.float32)
    ws = jnp.maximum(ws_ref[...], 1e-8)
    pre = jnp.maximum(hs + m / ws + b_ref[...], 0.0)
    y = x + pre
    mu = jnp.mean(y, axis=1, keepdims=True)
    var = jnp.mean((y - mu) ** 2, axis=1, keepdims=True)
    o_ref[...] = ((y - mu) * lax.rsqrt(var + 1e-5) * g_ref[...] + be_ref[...])


def _tc_dense(x2, acc, wsum, wst, wmt, bias, gamma, beta):
    R = 1000
    grid = (_N // R,)
    return pl.pallas_call(
        _tc_body,
        grid=grid,
        in_specs=[
            pl.BlockSpec((R, _D), lambda i: (i, 0)),
            pl.BlockSpec((R, _D), lambda i: (i, 0)),
            pl.BlockSpec((R, 1), lambda i: (i, 0)),
            pl.BlockSpec((_D, _D), lambda i: (0, 0)),
            pl.BlockSpec((_D, _D), lambda i: (0, 0)),
            pl.BlockSpec((1, _D), lambda i: (0, 0)),
            pl.BlockSpec((1, _D), lambda i: (0, 0)),
            pl.BlockSpec((1, _D), lambda i: (0, 0)),
        ],
        out_specs=pl.BlockSpec((R, _D), lambda i: (i, 0)),
        out_shape=jax.ShapeDtypeStruct((_N, _D), jnp.float32),
    )(x2, acc, wsum, wst, wmt, bias, gamma, beta)


def kernel(h, edge_index, edge_weight, W_self, W_msg, bias, gamma, beta):
    B, N, D = h.shape
    x2 = h.reshape(N, D)
    h_lo = x2[:, :_HALF]
    h_hi = x2[:, _HALF:]
    src = edge_index[0]
    dst = edge_index[1]

    accw, wsum2 = _sc_segment_sum(h_lo, h_hi, src, dst, edge_weight)
    acc = jnp.concatenate([accw[0], accw[1]], axis=1)
    wsum = wsum2.reshape(-1)[:N]

    out = _tc_dense(x2, acc, wsum.reshape(N, 1), W_self.T, W_msg.T,
                    bias.reshape(1, D), gamma.reshape(1, D),
                    beta.reshape(1, D))
    return out.reshape(B, N, D)


# speedup vs baseline: 5.9263x; 1.1539x over previous
import functools

import jax
import jax.numpy as jnp
from jax import lax
from jax.experimental import pallas as pl
from jax.experimental.pallas import tpu as pltpu
from jax.experimental.pallas import tpu_sc as plsc

_N = 10000
_D = 256
_E = 160000
_HALF = _D // 2
_NS = 16
_EP = _E // _NS
_CH = 80
_NCH = _EP // _CH
_GP = _CH // 16
_RBASE = 624
_ZROWS = 16
_WROWS = 80


def _sc_segment_sum(h_lo, h_hi, src, dst, ew):
    mesh = plsc.VectorSubcoreMesh(core_axis_name="c", subcore_axis_name="s")

    @functools.partial(
        pl.kernel,
        mesh=mesh,
        out_type=(
            jax.ShapeDtypeStruct((2, _N, _HALF), jnp.float32),
            jax.ShapeDtypeStruct((_WROWS, _HALF), jnp.float32),
        ),
        scratch_types=[
            pltpu.VMEM((_CH, _HALF), jnp.float32),
            pltpu.VMEM((_CH, _HALF), jnp.float32),
            pltpu.VMEM((_CH, _HALF), jnp.float32),
            pltpu.VMEM((_WROWS, _HALF), jnp.float32),
            pltpu.VMEM((_ZROWS, _HALF), jnp.float32),
            pltpu.VMEM((_CH,), jnp.float32),
            pltpu.VMEM((_CH,), jnp.float32),
            pltpu.VMEM((_CH,), jnp.float32),
            pltpu.VMEM((_CH,), jnp.int32),
            pltpu.VMEM((_CH,), jnp.int32),
            pltpu.VMEM((_CH,), jnp.int32),
            pltpu.VMEM((_CH,), jnp.int32),
            pltpu.VMEM((_CH,), jnp.int32),
            pltpu.VMEM((_CH,), jnp.int32),
            pltpu.VMEM((_CH,), jnp.int32),
            pltpu.VMEM((_CH,), jnp.int32),
            pltpu.VMEM((_CH,), jnp.int32),
            pltpu.VMEM((_WROWS,), jnp.int32),
            pltpu.SemaphoreType.DMA,
            pltpu.SemaphoreType.DMA,
            pltpu.SemaphoreType.DMA,
            pltpu.SemaphoreType.DMA,
            pltpu.SemaphoreType.DMA,
            pltpu.SemaphoreType.DMA,
            pltpu.SemaphoreType.DMA,
            pltpu.SemaphoreType.DMA,
            pltpu.SemaphoreType.DMA,
            pltpu.VMEM_SHARED((_N, _HALF), jnp.float32),
            pltpu.VMEM_SHARED((_WROWS, _HALF), jnp.float32),
        ],
    )
    def sc_kernel(hlo, hhi, srcx, dstx, eww, acc_out, ws_out,
                  rows0, rows1, rows2, wsl, zbuf,
                  wb0, wb1, wb2, srcb0, srcb1, srcb2,
                  dstb0, dstb1, dstb2, dstb3, dstb4, dstb5, iota_b,
                  sidx0, sidx1, sidx2, sg0, sg1, sg2, ss0, ss1, ss2,
                  acc_sh, ws_sh):
        rows = (rows0, rows1, rows2)
        wb = (wb0, wb1, wb2)
        srcb = (srcb0, srcb1, srcb2)
        dstb = (dstb0, dstb1, dstb2, dstb3, dstb4, dstb5)
        sidx = (sidx0, sidx1, sidx2)
        sg = (sg0, sg1, sg2)
        ss = (ss0, ss1, ss2)

        c = lax.axis_index("c")
        s = lax.axis_index("s")
        base_e = s * _EP
        row0 = pl.multiple_of(_RBASE * s + 16 * jnp.minimum(s, 1), 16)
        lanes = lax.iota(jnp.int32, 16)

        def _zw(i, carry):
            r = i // (_HALF // 16)
            j = i % (_HALF // 16)
            wsl[r, pl.ds(j * 16, 16)] = jnp.zeros((16,), jnp.float32)
            return carry
        lax.fori_loop(0, _WROWS * (_HALF // 16), _zw, 0)

        def _zb(i, carry):
            r = i // (_HALF // 16)
            j = i % (_HALF // 16)
            zbuf[r, pl.ds(j * 16, 16)] = jnp.zeros((16,), jnp.float32)
            return carry
        lax.fori_loop(0, _ZROWS * (_HALF // 16), _zb, 0)

        def _io(i, carry):
            iota_b[pl.ds(i * 16, 16)] = lanes + i * 16
            return carry
        lax.fori_loop(0, _WROWS // 16, _io, 0)

        for t in range(_RBASE // _ZROWS):
            pltpu.sync_copy(
                zbuf,
                acc_sh.at[pl.ds(pl.multiple_of(row0 + t * _ZROWS, 8),
                                _ZROWS)])

        @pl.when(s == 0)
        def _():
            pltpu.sync_copy(zbuf, acc_sh.at[pl.ds(_RBASE, 16)])
            for t in range(_WROWS // _ZROWS):
                pltpu.sync_copy(zbuf, ws_sh.at[pl.ds(t * _ZROWS, _ZROWS)])

        plsc.subcore_barrier()

        def issue_idx(k, sl, dl):
            off = base_e + k * _CH
            pltpu.async_copy(srcx.at[pl.ds(off, _CH)], srcb[sl], sidx[sl])
            pltpu.async_copy(dstx.at[pl.ds(off, _CH)], dstb[dl], sidx[sl])
            pltpu.async_copy(eww.at[pl.ds(off, _CH)], wb[sl], sidx[sl])

        def wait_idx(sl, dl):
            pltpu.make_async_copy(srcx.at[pl.ds(0, _CH)], srcb[sl],
                                  sidx[sl]).wait()
            pltpu.make_async_copy(dstx.at[pl.ds(0, _CH)], dstb[dl],
                                  sidx[sl]).wait()
            pltpu.make_async_copy(eww.at[pl.ds(0, _CH)], wb[sl],
                                  sidx[sl]).wait()

        def issue_gather(b):
            @pl.when(c == 0)
            def _():
                pltpu.async_copy(hlo.at[srcb[b]], rows[b], sg[b])

            @pl.when(c == 1)
            def _():
                pltpu.async_copy(hhi.at[srcb[b]], rows[b], sg[b])

        def wait_gather(b):
            pltpu.make_async_copy(hlo.at[srcb[b]], rows[b], sg[b]).wait()

        def issue_scatter(b, dl):
            pltpu.async_copy(rows[b], acc_sh.at[dstb[dl]], ss[b], add=True)

        def wait_scatter(b, dl):
            pltpu.make_async_copy(rows[b], acc_sh.at[dstb[dl]],
                                  ss[b]).wait()

        def compute(k, b, dl, wl_):
            def _sig(i, c2):
                v = wb[wl_][pl.ds(i * 16, 16)]
                wb[wl_][pl.ds(i * 16, 16)] = 1.0 / (1.0 + jnp.exp(-v))
                return c2
            lax.fori_loop(0, _CH // 16, _sig, 0)

            def _group(g, carry2):
                w16 = wb[wl_][pl.ds(g * 16, 16)]
                d16 = dstb[dl][pl.ds(g * 16, 16)]
                for lane in range(16):
                    wl = w16[lane]
                    e = g * 16 + lane
                    for j in range(_HALF // 16):
                        rows[b][e, pl.ds(j * 16, 16)] = (
                            rows[b][e, pl.ds(j * 16, 16)] * wl)
                    d = d16[lane]
                    woff = (d // 16) * 16
                    r = woff // _HALF
                    co = woff - r * _HALF
                    lv = d - woff
                    win = wsl[r, pl.ds(co, 16)]
                    wsl[r, pl.ds(co, 16)] = win + jnp.where(
                        lanes == lv, wl, jnp.float32(0.0))
                return carry2
            lax.fori_loop(0, _GP, _group, 0)

        def chunk_body(k, j):
            b = j % 3
            b2 = (j + 1) % 3
            wait_gather(b)
            wait_idx((j + 1) % 3, (j + 1) % 6)

            @pl.when(k >= 2)
            def _():
                wait_scatter(b2, (j + 4) % 6)
            issue_gather(b2)
            issue_idx(k + 2, (j + 2) % 3, (j + 2) % 6)
            compute(k, b, j % 6, j % 3)
            issue_scatter(b, j % 6)

        issue_idx(0, 0, 0)
        wait_idx(0, 0)
        issue_gather(0)
        issue_idx(1, 1, 1)

        def _hex(i, carry):
            for j in range(6):
                chunk_body(6 * i + j, j)
            return carry
        lax.fori_loop(0, _NCH // 6, _hex, 0)

        for k in range(_NCH - _NCH % 6, _NCH):
            j = k % 6
            b = j % 3
            b2 = (j + 1) % 3
            wait_gather(b)
            if k + 1 < _NCH:
                wait_idx((j + 1) % 3, (j + 1) % 6)
            wait_scatter(b2, (j + 4) % 6)
            if k + 1 < _NCH:
                issue_gather(b2)
            if k + 2 < _NCH:
                issue_idx(k + 2, (j + 2) % 3, (j + 2) % 6)
            compute(k, b, j % 6, j % 3)
            issue_scatter(b, j % 6)
        wait_scatter((_NCH - 2) % 3, (_NCH - 2) % 6)
        wait_scatter((_NCH - 1) % 3, (_NCH - 1) % 6)

        pltpu.sync_copy(wsl, ws_sh.at[iota_b], add=True)
        plsc.subcore_barrier()

        @pl.when(jnp.logical_and(c == 0, s == 0))
        def _():
            pltpu.sync_copy(ws_sh, ws_out)

        pltpu.sync_copy(acc_sh.at[pl.ds(row0, _RBASE)],
                        acc_out.at[c].at[pl.ds(row0, _RBASE)])

        @pl.when(s == 0)
        def _():
            pltpu.sync_copy(acc_sh.at[pl.ds(_RBASE, 16)],
                            acc_out.at[c].at[pl.ds(_RBASE, 16)])

    return sc_kernel(h_lo, h_hi, src, dst, ew)


def _tc_body(x_ref, a0_ref, a1_ref, ws_ref, wst_ref, wm0_ref, wm1_ref,
             b_ref, g_ref, be_ref, o_ref):
    x = x_ref[...]
    hs = jnp.dot(x, wst_ref[...], preferred_element_type=jnp.float32)
    m = (jnp.dot(a0_ref[...], wm0_ref[...], preferred_element_type=jnp.float32)
         + jnp.dot(a1_ref[...], wm1_ref[...],
                   preferred_element_type=jnp.float32))
    ws = jnp.maximum(ws_ref[...], 1e-8)
    pre = jnp.maximum(hs + m / ws + b_ref[...], 0.0)
    y = x + pre
    mu = jnp.mean(y, axis=1, keepdims=True)
    var = jnp.mean((y - mu) ** 2, axis=1, keepdims=True)
    o_ref[...] = ((y - mu) * lax.rsqrt(var + 1e-5) * g_ref[...] + be_ref[...])


def _tc_dense(x2, acc0, acc1, wsum, wst, wm0, wm1, bias, gamma, beta):
    R = 1000
    grid = (_N // R,)
    return pl.pallas_call(
        _tc_body,
        grid=grid,
        in_specs=[
            pl.BlockSpec((R, _D), lambda i: (i, 0)),
            pl.BlockSpec((R, _HALF), lambda i: (i, 0)),
            pl.BlockSpec((R, _HALF), lambda i: (i, 0)),
            pl.BlockSpec((R, 1), lambda i: (i, 0)),
            pl.BlockSpec((_D, _D), lambda i: (0, 0)),
            pl.BlockSpec((_HALF, _D), lambda i: (0, 0)),
            pl.BlockSpec((_HALF, _D), lambda i: (0, 0)),
            pl.BlockSpec((1, _D), lambda i: (0, 0)),
            pl.BlockSpec((1, _D), lambda i: (0, 0)),
            pl.BlockSpec((1, _D), lambda i: (0, 0)),
        ],
        out_specs=pl.BlockSpec((R, _D), lambda i: (i, 0)),
        out_shape=jax.ShapeDtypeStruct((_N, _D), jnp.float32),
    )(x2, acc0, acc1, wsum, wst, wm0, wm1, bias, gamma, beta)


def kernel(h, edge_index, edge_weight, W_self, W_msg, bias, gamma, beta):
    B, N, D = h.shape
    x2 = h.reshape(N, D)
    h_lo = x2[:, :_HALF]
    h_hi = x2[:, _HALF:]
    src = edge_index[0]
    dst = edge_index[1]

    accw, wsum2 = _sc_segment_sum(h_lo, h_hi, src, dst, edge_weight)
    wsum = wsum2.reshape(-1)[:N]
    wmt = W_msg.T

    out = _tc_dense(x2, accw[0], accw[1], wsum.reshape(N, 1), W_self.T,
                    wmt[:_HALF], wmt[_HALF:],
                    bias.reshape(1, D), gamma.reshape(1, D),
                    beta.reshape(1, D))
    return out.reshape(B, N, D)
